# baseline (speedup 1.0000x reference)
import numpy as np
import jax
import jax.numpy as jnp
from jax.experimental import pallas as pl

_S = 512
_C = 6
_P = 8
_B = _S // _P


def _body(ids_ref, x_ref, W_ref, grp_ref, row_ref, bias_ref, E_ref, out_ref):
    i = pl.program_id(0)
    nb = pl.num_programs(0)

    @pl.when(i == 0)
    def _init():
        out_ref[...] = jnp.zeros_like(out_ref)

    x = x_ref[...]
    y = jax.lax.dot_general(x.astype(jnp.bfloat16), W_ref[...],
                            (((1,), (1,)), ((), ())),
                            preferred_element_type=jnp.float32)
    r = y.shape[0]
    col = jax.lax.broadcasted_iota(jnp.int32, (r, _P), 1)
    y = y + (col == _C).astype(jnp.float32)

    ids = ids_ref[0, 0, :]
    a = ids & (_P - 1)
    bhi = ids >> 3

    y8 = jnp.concatenate([y] * _P, axis=1)
    U = jnp.where(a[:, None] == grp_ref[...], y8, 0.0).astype(jnp.bfloat16)
    OHb = (row_ref[...] == bhi[None, :]).astype(jnp.bfloat16)
    out_ref[...] += jax.lax.dot_general(OHb, U, (((1,), (0,)), ((), ())),
                                        preferred_element_type=jnp.float32)

    @pl.when(i == nb - 1)
    def _fin():
        acc = out_ref[...]
        cnt = jax.lax.dot_general(acc, E_ref[...], (((1,), (0,)), ((), ())),
                                  preferred_element_type=jnp.float32)
        z = acc / jnp.clip(cnt, 1.0, None) + bias_ref[...]
        out_ref[...] = jax.nn.sigmoid(z)


def kernel(x, batch, W, b):
    n, d = x.shape
    r = 0
    for cand in range(128, 4097, 128):
        if n % cand == 0:
            r = cand
    if r == 0:
        for cand in range(8, 4097, 8):
            if n % cand == 0:
                r = cand
    nb = n // r

    ids = batch.astype(jnp.int32).reshape(nb, 1, r)
    Wp = jnp.zeros((_P, d), jnp.bfloat16).at[:_C].set(W.astype(jnp.bfloat16))
    nl = _P * _P
    grp = (np.arange(nl, dtype=np.int32) // _P).reshape(1, nl)
    row = np.arange(_B, dtype=np.int32).reshape(_B, 1)
    bias = jnp.tile(jnp.concatenate([b, jnp.zeros((_P - _C,), b.dtype)]),
                    (_P,)).reshape(1, nl)
    jj = np.arange(nl)
    E = ((jj[:, None] // _P == jj[None, :] // _P) &
         (jj[:, None] % _P == _C)).astype(np.float32)

    out = pl.pallas_call(
        _body,
        grid=(nb,),
        in_specs=[
            pl.BlockSpec((1, 1, r), lambda i: (i, 0, 0)),
            pl.BlockSpec((r, d), lambda i: (i, 0)),
            pl.BlockSpec((_P, d), lambda i: (0, 0)),
            pl.BlockSpec((1, nl), lambda i: (0, 0)),
            pl.BlockSpec((_B, 1), lambda i: (0, 0)),
            pl.BlockSpec((1, nl), lambda i: (0, 0)),
            pl.BlockSpec((nl, nl), lambda i: (0, 0)),
        ],
        out_specs=pl.BlockSpec((_B, nl), lambda i: (0, 0)),
        out_shape=jax.ShapeDtypeStruct((_B, nl), jnp.float32),
    )(ids, x, Wp, jnp.asarray(grp), jnp.asarray(row), bias, jnp.asarray(E))
    return out.reshape(_S, _P)[:, :_C]


# speedup vs baseline: 2.0858x; 2.0858x over previous
import numpy as np
import jax
import jax.numpy as jnp
from jax.experimental import pallas as pl

_S = 512
_C = 6
_P = 8
_B = _S // _P


def _body(ids_ref, x_ref, W_ref, grp_ref, row_ref, ones_ref, bias_ref, E_ref,
          out_ref):
    i = pl.program_id(0)
    nb = pl.num_programs(0)

    @pl.when(i == 0)
    def _init():
        out_ref[...] = jnp.zeros_like(out_ref)

    x = x_ref[...]
    y8 = jax.lax.dot_general(x.astype(jnp.bfloat16), W_ref[...],
                             (((1,), (0,)), ((), ())),
                             preferred_element_type=jnp.float32)
    y8 = y8 + ones_ref[...]

    ids = ids_ref[0, 0, :]
    a = ids & (_P - 1)
    bhi = ids >> 3

    U = jnp.where(a[:, None] == grp_ref[...], y8, 0.0).astype(jnp.bfloat16)
    OHb = (row_ref[...] == bhi[None, :]).astype(jnp.bfloat16)
    out_ref[...] += jax.lax.dot_general(OHb, U, (((1,), (0,)), ((), ())),
                                        preferred_element_type=jnp.float32)

    @pl.when(i == nb - 1)
    def _fin():
        acc = out_ref[...]
        cnt = jax.lax.dot_general(acc, E_ref[...], (((1,), (0,)), ((), ())),
                                  preferred_element_type=jnp.float32)
        z = acc / jnp.clip(cnt, 1.0, None) + bias_ref[...]
        out_ref[...] = jax.nn.sigmoid(z)


def kernel(x, batch, W, b):
    n, d = x.shape
    r = 0
    for cand in range(128, 4097, 128):
        if n % cand == 0:
            r = cand
    if r == 0:
        for cand in range(8, 4097, 8):
            if n % cand == 0:
                r = cand
    nb = n // r

    ids = batch.astype(jnp.int32).reshape(nb, 1, r)
    nl = _P * _P
    Wp = jnp.zeros((_P, d), W.dtype).at[:_C].set(W)
    W64 = jnp.tile(Wp.T, (1, _P)).astype(jnp.bfloat16)
    grp = (np.arange(nl, dtype=np.int32) // _P).reshape(1, nl)
    row = np.arange(_B, dtype=np.int32).reshape(_B, 1)
    ones = (np.arange(nl) % _P == _C).astype(np.float32).reshape(1, nl)
    bias = jnp.tile(jnp.concatenate([b, jnp.zeros((_P - _C,), b.dtype)]),
                    (_P,)).reshape(1, nl)
    jj = np.arange(nl)
    E = ((jj[:, None] // _P == jj[None, :] // _P) &
         (jj[:, None] % _P == _C)).astype(np.float32)

    out = pl.pallas_call(
        _body,
        grid=(nb,),
        in_specs=[
            pl.BlockSpec((1, 1, r), lambda i: (i, 0, 0)),
            pl.BlockSpec((r, d), lambda i: (i, 0)),
            pl.BlockSpec((d, nl), lambda i: (0, 0)),
            pl.BlockSpec((1, nl), lambda i: (0, 0)),
            pl.BlockSpec((_B, 1), lambda i: (0, 0)),
            pl.BlockSpec((1, nl), lambda i: (0, 0)),
            pl.BlockSpec((1, nl), lambda i: (0, 0)),
            pl.BlockSpec((nl, nl), lambda i: (0, 0)),
        ],
        out_specs=pl.BlockSpec((_B, nl), lambda i: (0, 0)),
        out_shape=jax.ShapeDtypeStruct((_B, nl), jnp.float32),
    )(ids, x, W64, jnp.asarray(grp), jnp.asarray(row), jnp.asarray(ones),
      bias, jnp.asarray(E))
    return out.reshape(_S, _P)[:, :_C]
